# baseline (speedup 1.0000x reference)
import functools

import jax
import jax.numpy as jnp
from jax import lax
from jax.experimental import pallas as pl
from jax.experimental.pallas import tpu as pltpu
from jax.experimental.pallas import tpu_sc as plsc

D = 64
DP = 128
B = 16384
NC = 2
NS = 16
NW = NC * NS
BPW = B // NW
BLK = 128
NBLK = BPW // BLK
CH = 8
NCH = BLK // CH
EG = 125000
RG = 125


def _make_kernel():
    mesh = plsc.VectorSubcoreMesh(core_axis_name="c", subcore_axis_name="s")

    slab = lambda: pltpu.VMEM((CH, 8, DP), jnp.float32)

    @functools.partial(
        pl.kernel,
        mesh=mesh,
        out_type=jax.ShapeDtypeStruct((B,), jnp.float32),
        compiler_params=pltpu.CompilerParams(needs_layout_passes=False),
        scratch_types=[
            pltpu.VMEM((BLK,), jnp.int32),
            pltpu.VMEM((BLK,), jnp.int32),
            pltpu.VMEM((BLK,), jnp.int32),
            pltpu.VMEM((BLK,), jnp.int32),
            pltpu.VMEM((BLK,), jnp.int32),
            pltpu.VMEM((BLK,), jnp.int32),
            [[slab() for _ in range(6)] for _ in range(2)],
            pltpu.VMEM((BPW * 16,), jnp.float32),
            pltpu.VMEM((BPW,), jnp.float32),
            pltpu.SemaphoreType.DMA,
            pltpu.SemaphoreType.DMA,
        ],
    )
    def complex_score(head, relation, tail, ent_r, ent_i, rel_r, rel_i,
                      out, ihv, irv, itv, gh, gr, gt,
                      bufs, stage, out_v, sem0, sem1):
        wid = lax.axis_index("s") * NC + lax.axis_index("c")
        base = wid * BPW
        sems = (sem0, sem1)
        lane16 = lax.iota(jnp.int32, 16) * 16

        def srcs(c, slot):
            del slot
            s = pl.ds(c * CH, CH)
            return (
                (ent_r.at[gh.at[s]], 0), (ent_i.at[gh.at[s]], 1),
                (ent_r.at[gt.at[s]], 2), (ent_i.at[gt.at[s]], 3),
                (rel_r.at[gr.at[s]], 4), (rel_i.at[gr.at[s]], 5),
            )

        def fire(c, slot):
            for src, t in srcs(c, slot):
                pltpu.async_copy(src, bufs[slot][t], sems[slot])

        def drain(c, slot):
            for src, t in srcs(c, slot):
                pltpu.make_async_copy(src, bufs[slot][t], sems[slot]).wait()

        def compute(blk, c, slot, idsh, idsr, idst, lo):
            for u in range(CH):
                i = c * CH + u
                sh = idsh[lo + u] & 7
                st = idst[lo + u] & 7
                sq = idsr[lo + u] & 7
                acc = jnp.zeros((16,), jnp.float32)
                for k in range(D // 16):
                    s = pl.ds(k * 16, 16)
                    hrv = bufs[slot][0][u, sh, s]
                    hiv = bufs[slot][1][u, sh, s]
                    trv = bufs[slot][2][u, st, s]
                    tiv = bufs[slot][3][u, st, s]
                    rrv = bufs[slot][4][u, sq, s]
                    riv = bufs[slot][5][u, sq, s]
                    a = hrv * trv - hiv * tiv
                    bb = hrv * tiv + hiv * trv
                    acc = acc + rrv * a + riv * bb
                stage[pl.ds((blk * BLK + i) * 16, 16)] = acc

        def block(blk, _):
            off = pl.multiple_of(base + blk * BLK, 8)
            pltpu.sync_copy(head.at[pl.ds(off, BLK)], ihv)
            pltpu.sync_copy(relation.at[pl.ds(off, BLK)], irv)
            pltpu.sync_copy(tail.at[pl.ds(off, BLK)], itv)
            for s8 in range(BLK // 16):
                s = pl.ds(s8 * 16, 16)
                gh[s] = ihv[s] >> 3
                gr[s] = irv[s] >> 3
                gt[s] = itv[s] >> 3
            fire(0, 0)

            def step(t, _):
                c = t * 2
                s16 = pl.ds(t * 16, 16)
                idsh = ihv[s16]
                idsr = irv[s16]
                idst = itv[s16]
                fire(c + 1, 1)
                drain(c, 0)
                compute(blk, c, 0, idsh, idsr, idst, 0)

                @pl.when(c + 2 < NCH)
                def _():
                    fire(c + 2, 0)

                drain(c + 1, 1)
                compute(blk, c + 1, 1, idsh, idsr, idst, CH)
                return 0

            lax.fori_loop(0, NCH // 2, step, 0)
            return 0

        lax.fori_loop(0, NBLK, block, 0)

        def group(g, _):
            gbase = g * 256
            tot = jnp.zeros((16,), jnp.float32)
            for j in range(16):
                tot = tot + plsc.load_gather(stage, [gbase + lane16 + j])
            out_v[pl.ds(g * 16, 16)] = tot
            return 0

        lax.fori_loop(0, BPW // 16, group, 0)
        pltpu.sync_copy(out_v, out.at[pl.ds(base, BPW)])

    return complex_score


_KERNEL = _make_kernel()


def kernel(head, relation, tail, entity_real, entity_imag,
           relation_real, relation_imag):
    pad = ((0, 0), (0, DP - D))
    ent_r = jnp.pad(entity_real, pad).reshape(EG, 8, DP)
    ent_i = jnp.pad(entity_imag, pad).reshape(EG, 8, DP)
    rel_r = jnp.pad(relation_real, pad).reshape(RG, 8, DP)
    rel_i = jnp.pad(relation_imag, pad).reshape(RG, 8, DP)
    return _KERNEL(head, relation, tail, ent_r, ent_i, rel_r, rel_i)


# speedup vs baseline: 1.1727x; 1.1727x over previous
import functools

import jax
import jax.numpy as jnp
from jax import lax
from jax.experimental import pallas as pl
from jax.experimental.pallas import tpu as pltpu
from jax.experimental.pallas import tpu_sc as plsc

D = 64
DP = 128
B = 16384
NC = 2
NS = 16
NW = NC * NS
BPW = B // NW
C = 64
NCHUNK = BPW // C


def _make_kernel():
    mesh = plsc.VectorSubcoreMesh(core_axis_name="c", subcore_axis_name="s")

    @functools.partial(
        pl.kernel,
        mesh=mesh,
        out_type=jax.ShapeDtypeStruct((B,), jnp.float32),
        compiler_params=pltpu.CompilerParams(needs_layout_passes=False),
        scratch_types=[
            [pltpu.VMEM((C,), jnp.int32) for _ in range(2)],
            [pltpu.VMEM((C,), jnp.int32) for _ in range(2)],
            [pltpu.VMEM((C,), jnp.int32) for _ in range(2)],
            [[pltpu.VMEM((C, DP), jnp.float32) for _ in range(6)]
             for _ in range(2)],
            pltpu.VMEM((C * 16,), jnp.float32),
            pltpu.VMEM((BPW,), jnp.float32),
            pltpu.SemaphoreType.DMA,
            pltpu.SemaphoreType.DMA,
        ],
    )
    def complex_score(head, relation, tail, ent_r, ent_i, rel_r, rel_i,
                      out, ih, ir, it, bufs, stage, out_v, sem0, sem1):
        wid = lax.axis_index("s") * NC + lax.axis_index("c")
        base = wid * BPW
        sems = (sem0, sem1)
        lane16 = lax.iota(jnp.int32, 16) * 16

        def load_idx(c, slot):
            off = pl.multiple_of(base + c * C, 8)
            pltpu.sync_copy(head.at[pl.ds(off, C)], ih[slot])
            pltpu.sync_copy(relation.at[pl.ds(off, C)], ir[slot])
            pltpu.sync_copy(tail.at[pl.ds(off, C)], it[slot])

        def srcs(slot):
            return ((ent_r.at[ih[slot]], 0), (ent_i.at[ih[slot]], 1),
                    (ent_r.at[it[slot]], 2), (ent_i.at[it[slot]], 3),
                    (rel_r.at[ir[slot]], 4), (rel_i.at[ir[slot]], 5))

        def fire(slot):
            for src, t in srcs(slot):
                pltpu.async_copy(src, bufs[slot][t], sems[slot])

        def drain(slot):
            for src, t in srcs(slot):
                pltpu.make_async_copy(src, bufs[slot][t], sems[slot]).wait()

        def compute(slot):
            def row(i, _):
                acc = jnp.zeros((16,), jnp.float32)
                for k in range(D // 16):
                    s = pl.ds(k * 16, 16)
                    hrv = bufs[slot][0][i, s]
                    hiv = bufs[slot][1][i, s]
                    trv = bufs[slot][2][i, s]
                    tiv = bufs[slot][3][i, s]
                    rrv = bufs[slot][4][i, s]
                    riv = bufs[slot][5][i, s]
                    a = hrv * trv - hiv * tiv
                    bb = hrv * tiv + hiv * trv
                    acc = acc + rrv * a + riv * bb
                stage[pl.ds(i * 16, 16)] = acc
                return 0

            lax.fori_loop(0, C, row, 0)

        def reduce_out(c):
            def group(g, _):
                gbase = g * 256
                tot = jnp.zeros((16,), jnp.float32)
                for j in range(16):
                    tot = tot + plsc.load_gather(stage,
                                                 [gbase + lane16 + j])
                out_v[pl.ds(c * C + g * 16, 16)] = tot
                return 0

            lax.fori_loop(0, C // 16, group, 0)

        load_idx(0, 0)
        fire(0)
        for c in range(NCHUNK):
            slot = c % 2
            if c + 1 < NCHUNK:
                load_idx(c + 1, 1 - slot)
                fire(1 - slot)
            drain(slot)
            compute(slot)
            reduce_out(c)
        pltpu.sync_copy(out_v, out.at[pl.ds(base, BPW)])

    return complex_score


_KERNEL = _make_kernel()


def kernel(head, relation, tail, entity_real, entity_imag,
           relation_real, relation_imag):
    pad = ((0, 0), (0, DP - D))
    ent_r = jnp.pad(entity_real, pad)
    ent_i = jnp.pad(entity_imag, pad)
    rel_r = jnp.pad(relation_real, pad)
    rel_i = jnp.pad(relation_imag, pad)
    return _KERNEL(head, relation, tail, ent_r, ent_i, rel_r, rel_i)
